# baseline (speedup 1.0000x reference)
import jax
import jax.numpy as jnp
from jax import lax
from jax.experimental import pallas as pl
from jax.experimental.pallas import tpu as pltpu

_C = 64
_NSEM = 4


def kernel(node_idx, table):
    B = node_idx.shape[0]
    V, D = table.shape
    n_chunks = B // _C
    idx = node_idx.astype(jnp.int32)

    def body(idx_smem, table_hbm, out_hbm, rows_v, sems, out_sem):
        def start_chunk(c):
            s = lax.rem(c, _NSEM)
            base = c * _C
            for l in range(_C):
                r = idx_smem[base + l]
                pltpu.make_async_copy(
                    table_hbm.at[pl.ds(r, 1)],
                    rows_v.at[pl.ds(base + l, 1)],
                    sems.at[s]).start()

        def wait_chunk(c):
            pltpu.make_async_copy(
                table_hbm.at[pl.ds(0, _C)],
                rows_v.at[pl.ds(c * _C, _C)],
                sems.at[lax.rem(c, _NSEM)]).wait()

        def step(c, carry):
            start_chunk(c)

            @pl.when(c >= _NSEM - 1)
            def _():
                wait_chunk(c - (_NSEM - 1))

            return carry

        lax.fori_loop(0, n_chunks, step, 0)

        def drain(c, carry):
            wait_chunk(c)
            return carry

        lax.fori_loop(n_chunks - (_NSEM - 1), n_chunks, drain, 0)

        cp = pltpu.make_async_copy(rows_v, out_hbm, out_sem)
        cp.start()
        cp.wait()

    return pl.pallas_call(
        body,
        out_shape=jax.ShapeDtypeStruct((B, D), jnp.float32),
        in_specs=[
            pl.BlockSpec(memory_space=pltpu.SMEM),
            pl.BlockSpec(memory_space=pl.ANY),
        ],
        out_specs=pl.BlockSpec(memory_space=pl.ANY),
        scratch_shapes=[
            pltpu.VMEM((B, D), jnp.float32),
            pltpu.SemaphoreType.DMA((_NSEM,)),
            pltpu.SemaphoreType.DMA,
        ],
    )(idx, table)


# speedup vs baseline: 1.0330x; 1.0330x over previous
import functools

import jax
import jax.numpy as jnp
from jax import lax
from jax.experimental import pallas as pl
from jax.experimental.pallas import tpu as pltpu
from jax.experimental.pallas import tpu_sc as plsc

_SC_ROWS = 8192
_GROUPS_PER_BATCH = 8
_C = 64
_NSEM = 4


def _sc_gather(idx2, table, b_per_w, nc, ns):
    nw = nc * ns
    n_groups = b_per_w // 16
    n_batches = n_groups // _GROUPS_PER_BATCH
    rows_per_batch = _GROUPS_PER_BATCH * 16
    D = table.shape[1]

    mesh = plsc.VectorSubcoreMesh(core_axis_name="c", subcore_axis_name="s")

    @functools.partial(
        pl.kernel,
        mesh=mesh,
        out_type=jax.ShapeDtypeStruct((nw * b_per_w, D), jnp.float32),
        scratch_types=[
            pltpu.VMEM((b_per_w,), jnp.int32),
            pltpu.VMEM((b_per_w, D), jnp.float32),
            pltpu.SemaphoreType.DMA,
        ],
    )
    def body(idx_hbm, table_hbm, out_hbm, idx_v, rows_v, sem):
        wid = lax.axis_index("s") * nc + lax.axis_index("c")
        base = wid * b_per_w
        pltpu.sync_copy(idx_hbm.at[wid], idx_v)

        def fire_group(g, carry):
            vec = idx_v[pl.ds(g * 16, 16)]
            for l in range(16):
                r = vec[l]
                pltpu.async_copy(table_hbm.at[pl.ds(r, 1)],
                                 rows_v.at[pl.ds(g * 16 + l, 1)], sem)
            return carry

        for f in range(n_batches):
            lax.fori_loop(f * _GROUPS_PER_BATCH, (f + 1) * _GROUPS_PER_BATCH,
                          fire_group, 0)
            pltpu.make_async_copy(
                table_hbm.at[pl.ds(0, rows_per_batch)],
                rows_v.at[pl.ds(f * rows_per_batch, rows_per_batch)],
                sem).wait()

        pltpu.sync_copy(rows_v, out_hbm.at[pl.ds(base, b_per_w)])

    return body(idx2, table)


def _tc_gather(idx_tc, table):
    n = idx_tc.shape[0]
    D = table.shape[1]
    n_chunks = n // _C

    def body(idx_smem, table_hbm, out_hbm, rows_v, sems, out_sem):
        def start_chunk(c):
            s = lax.rem(c, _NSEM)
            base = c * _C
            for l in range(_C):
                r = idx_smem[base + l]
                pltpu.make_async_copy(
                    table_hbm.at[pl.ds(r, 1)],
                    rows_v.at[pl.ds(base + l, 1)],
                    sems.at[s]).start()

        def wait_chunk(c):
            pltpu.make_async_copy(
                table_hbm.at[pl.ds(0, _C)],
                rows_v.at[pl.ds(c * _C, _C)],
                sems.at[lax.rem(c, _NSEM)]).wait()

        def step(c, carry):
            start_chunk(c)

            @pl.when(c >= _NSEM - 1)
            def _():
                wait_chunk(c - (_NSEM - 1))

            return carry

        lax.fori_loop(0, n_chunks, step, 0)

        def drain(c, carry):
            wait_chunk(c)
            return carry

        lax.fori_loop(n_chunks - (_NSEM - 1), n_chunks, drain, 0)

        cp = pltpu.make_async_copy(rows_v, out_hbm, out_sem)
        cp.start()
        cp.wait()

    return pl.pallas_call(
        body,
        out_shape=jax.ShapeDtypeStruct((n, D), jnp.float32),
        in_specs=[
            pl.BlockSpec(memory_space=pltpu.SMEM),
            pl.BlockSpec(memory_space=pl.ANY),
        ],
        out_specs=pl.BlockSpec(memory_space=pl.ANY),
        scratch_shapes=[
            pltpu.VMEM((n, D), jnp.float32),
            pltpu.SemaphoreType.DMA((_NSEM,)),
            pltpu.SemaphoreType.DMA,
        ],
    )(idx_tc, table)


def kernel(node_idx, table):
    info = plsc.get_sparse_core_info()
    nc, ns = info.num_cores, info.num_subcores
    nw = nc * ns
    b_per_w = _SC_ROWS // nw

    idx = node_idx.astype(jnp.int32)
    idx_sc = idx[:_SC_ROWS].reshape(nw, b_per_w)
    idx_tc = idx[_SC_ROWS:]

    out_sc = _sc_gather(idx_sc, table, b_per_w, nc, ns)
    out_tc = _tc_gather(idx_tc, table)
    return jnp.concatenate([out_sc, out_tc], axis=0)
